# Initial kernel scaffold; baseline (speedup 1.0000x reference)
import jax
import jax.numpy as jnp
from jax.experimental import pallas as pl


def kernel(embeddings, confidence, target, triplets):
    raise NotImplementedError("write your pallas kernel here")


# speedup vs baseline: 6.0762x; 6.0762x over previous
import functools

import jax
import jax.numpy as jnp
from jax import lax
from jax.experimental import pallas as pl
from jax.experimental.pallas import tpu as pltpu
from jax.experimental.pallas import tpu_sc as plsc

MARGIN1 = 0.4
MARGIN2 = 0.4

NC = 2
NS = 16
LANES = 16
NW = NC * NS


def _make_lse(B, C, RB):
    def body(c_ref, o_ref):
        x = c_ref[...]
        m = jnp.max(x, axis=1)
        s = jnp.sum(jnp.exp(x - m[:, None]), axis=1)
        o_ref[...] = m + jnp.log(s)

    return pl.pallas_call(
        body,
        grid=(B // RB,),
        in_specs=[pl.BlockSpec((RB, C), lambda i: (i, 0))],
        out_specs=pl.BlockSpec((RB,), lambda i: (i,)),
        out_shape=jax.ShapeDtypeStruct((B,), jnp.float32),
    )


def _make_sc_main(B, C, D, T, K):
    TW = T // NW
    NCHUNK = TW // K
    GROUPS = K // LANES

    mesh = plsc.VectorSubcoreMesh(core_axis_name="c", subcore_axis_name="s")

    @functools.partial(
        pl.kernel,
        mesh=mesh,
        compiler_params=pltpu.CompilerParams(
            needs_layout_passes=False, use_tc_tiling_on_sc=False),
        out_type=jax.ShapeDtypeStruct((NW, LANES), jnp.float32),
        scratch_types=[
            pltpu.VMEM((B,), jnp.int32),
            pltpu.VMEM((B,), jnp.float32),
            pltpu.VMEM((K,), jnp.int32),
            pltpu.VMEM((K,), jnp.int32),
            pltpu.VMEM((K,), jnp.int32),
            pltpu.VMEM((K,), jnp.int32),
            pltpu.VMEM((K, D), jnp.float32),
            pltpu.VMEM((K, D), jnp.float32),
            pltpu.VMEM((K, D), jnp.float32),
            pltpu.VMEM((K, D), jnp.float32),
            pltpu.VMEM((4 * K,), jnp.int32),
            pltpu.VMEM((4 * K,), jnp.float32),
            pltpu.VMEM((LANES,), jnp.float32),
            pltpu.SemaphoreType.DMA,
        ],
    )
    def sc_main(emb, aidx, pidx, ridx, nidx, tgt, lse, conf_flat, out,
                target_v, lse_v, idxa_v, idxp_v, idxr_v, idxn_v,
                ra_v, rp_v, rr_v, rn_v, fidx_v, cval_v, accv, sem):
        wid = lax.axis_index("s") * NC + lax.axis_index("c")
        base_t = wid * TW

        pltpu.sync_copy(tgt, target_v)
        pltpu.sync_copy(lse, lse_v)

        zf = jnp.zeros((LANES,), jnp.float32)
        iota = lax.iota(jnp.int32, LANES)

        def chunk_body(ch, acc):
            tb = base_t + ch * K
            pltpu.sync_copy(aidx.at[pl.ds(tb, K)], idxa_v)
            pltpu.sync_copy(pidx.at[pl.ds(tb, K)], idxp_v)
            pltpu.sync_copy(ridx.at[pl.ds(tb, K)], idxr_v)
            pltpu.sync_copy(nidx.at[pl.ds(tb, K)], idxn_v)

            cps = [
                pltpu.async_copy(emb.at[idxa_v], ra_v, sem),
                pltpu.async_copy(emb.at[idxp_v], rp_v, sem),
                pltpu.async_copy(emb.at[idxr_v], rr_v, sem),
                pltpu.async_copy(emb.at[idxn_v], rn_v, sem),
            ]

            def group_a(g, carry):
                gb = g * LANES
                av = idxa_v[pl.ds(gb, LANES)]
                rv = idxr_v[pl.ds(gb, LANES)]
                nv = idxn_v[pl.ds(gb, LANES)]
                ta = plsc.load_gather(target_v, [av])
                tr = plsc.load_gather(target_v, [rv])
                tn = plsc.load_gather(target_v, [nv])
                fidx_v[pl.ds(gb, LANES)] = av * C + tr
                fidx_v[pl.ds(K + gb, LANES)] = rv * C + ta
                fidx_v[pl.ds(2 * K + gb, LANES)] = av * C + tn
                fidx_v[pl.ds(3 * K + gb, LANES)] = nv * C + ta
                return carry

            lax.fori_loop(0, GROUPS, group_a, 0)

            cpc = pltpu.async_copy(conf_flat.at[fidx_v], cval_v, sem)
            for cp in cps:
                cp.wait()
            cpc.wait()

            def group_b(g, acc_in):
                gb = g * LANES
                jv = gb + iota
                av = idxa_v[pl.ds(gb, LANES)]
                rv = idxr_v[pl.ds(gb, LANES)]
                nv = idxn_v[pl.ds(gb, LANES)]
                la = plsc.load_gather(lse_v, [av])
                lr = plsc.load_gather(lse_v, [rv])
                ln = plsc.load_gather(lse_v, [nv])
                c1 = cval_v[pl.ds(gb, LANES)]
                c2 = cval_v[pl.ds(K + gb, LANES)]
                c3 = cval_v[pl.ds(2 * K + gb, LANES)]
                c4 = cval_v[pl.ds(3 * K + gb, LANES)]
                w_rel = jnp.exp(jnp.exp(c1 - la) + jnp.exp(c2 - lr))
                w_neg = jnp.exp(jnp.exp(c3 - la) + jnp.exp(c4 - ln))

                def dbody(dd, c):
                    dap, dar, dan, dv = c
                    for _ in range(4):
                        ea = plsc.load_gather(ra_v, [jv, dv])
                        ep = plsc.load_gather(rp_v, [jv, dv])
                        er = plsc.load_gather(rr_v, [jv, dv])
                        en = plsc.load_gather(rn_v, [jv, dv])
                        s1 = ea - ep
                        s2 = ea - er
                        s3 = ea - en
                        dap = dap + s1 * s1
                        dar = dar + s2 * s2
                        dan = dan + s3 * s3
                        dv = dv + 1
                    return (dap, dar, dan, dv)

                dap, dar, dan, _ = lax.fori_loop(
                    0, D // 4, dbody,
                    (zf, zf, zf, jnp.zeros((LANES,), jnp.int32)))
                loss = (jnp.maximum(dap - dar + w_rel * MARGIN1, 0.0)
                        + jnp.maximum(dar - dan + w_neg * MARGIN2, 0.0))
                return acc_in + loss

            return lax.fori_loop(0, GROUPS, group_b, acc)

        acc = lax.fori_loop(0, NCHUNK, chunk_body, zf)
        accv[...] = acc
        pltpu.sync_copy(accv, out.at[wid])

    return sc_main


def kernel(embeddings, confidence, target, triplets):
    B, D = embeddings.shape
    C = confidence.shape[1]
    T = triplets.shape[0]

    lse = _make_lse(B, C, 512)(confidence)

    a_idx = triplets[:, 0]
    p_idx = triplets[:, 1]
    r_idx = triplets[:, 2]
    n_idx = triplets[:, 3]
    conf_flat = confidence.reshape(-1)

    partials = _make_sc_main(B, C, D, T, 256)(
        embeddings, a_idx, p_idx, r_idx, n_idx, target, lse, conf_flat)
    mean = jnp.sum(partials) / jnp.float32(T)
    return (mean, jnp.asarray(T, dtype=jnp.int32))


# speedup vs baseline: 10.6241x; 1.7485x over previous
import functools

import jax
import jax.numpy as jnp
from jax import lax
from jax.experimental import pallas as pl
from jax.experimental.pallas import tpu as pltpu
from jax.experimental.pallas import tpu_sc as plsc

MARGIN1 = 0.4
MARGIN2 = 0.4

NC = 2
NS = 16
LANES = 16
NW = NC * NS


def _make_lse(B, C, RB):
    def body(c_ref, o_ref):
        x = c_ref[...]
        m = jnp.max(x, axis=1)
        s = jnp.sum(jnp.exp(x - m[:, None]), axis=1)
        o_ref[...] = m + jnp.log(s)

    return pl.pallas_call(
        body,
        grid=(B // RB,),
        in_specs=[pl.BlockSpec((RB, C), lambda i: (i, 0))],
        out_specs=pl.BlockSpec((RB,), lambda i: (i,)),
        out_shape=jax.ShapeDtypeStruct((B,), jnp.float32),
    )


def _make_sc_main(B, C, D, T, K):
    TW = T // NW
    NCHUNK = TW // K
    GROUPS = K // LANES

    mesh = plsc.VectorSubcoreMesh(core_axis_name="c", subcore_axis_name="s")

    @functools.partial(
        pl.kernel,
        mesh=mesh,
        compiler_params=pltpu.CompilerParams(
            needs_layout_passes=False, use_tc_tiling_on_sc=False),
        out_type=jax.ShapeDtypeStruct((NW, LANES), jnp.float32),
        scratch_types=[
            pltpu.VMEM((B,), jnp.int32),
            pltpu.VMEM((B,), jnp.float32),
            pltpu.VMEM((K,), jnp.int32),
            pltpu.VMEM((K,), jnp.int32),
            pltpu.VMEM((K,), jnp.int32),
            pltpu.VMEM((K,), jnp.int32),
            pltpu.VMEM((K, D), jnp.float32),
            pltpu.VMEM((K, D), jnp.float32),
            pltpu.VMEM((K, D), jnp.float32),
            pltpu.VMEM((K, D), jnp.float32),
            pltpu.VMEM((4 * K,), jnp.int32),
            pltpu.VMEM((4 * K,), jnp.float32),
            pltpu.VMEM((LANES,), jnp.float32),
            pltpu.SemaphoreType.DMA,
        ],
    )
    def sc_main(emb, aidx, pidx, ridx, nidx, tgt, lse, conf_flat, out,
                target_v, lse_v, idxa_v, idxp_v, idxr_v, idxn_v,
                ra_v, rp_v, rr_v, rn_v, fidx_v, cval_v, accv, sem):
        wid = lax.axis_index("s") * NC + lax.axis_index("c")
        base_t = wid * TW

        pltpu.sync_copy(tgt, target_v)
        pltpu.sync_copy(lse, lse_v)

        zf = jnp.zeros((LANES,), jnp.float32)
        iota = lax.iota(jnp.int32, LANES)

        def chunk_body(ch, acc):
            tb = base_t + ch * K
            pltpu.sync_copy(aidx.at[pl.ds(tb, K)], idxa_v)
            pltpu.sync_copy(pidx.at[pl.ds(tb, K)], idxp_v)
            pltpu.sync_copy(ridx.at[pl.ds(tb, K)], idxr_v)
            pltpu.sync_copy(nidx.at[pl.ds(tb, K)], idxn_v)

            cps = [
                pltpu.async_copy(emb.at[idxa_v], ra_v, sem),
                pltpu.async_copy(emb.at[idxp_v], rp_v, sem),
                pltpu.async_copy(emb.at[idxr_v], rr_v, sem),
                pltpu.async_copy(emb.at[idxn_v], rn_v, sem),
            ]

            def group_a(g, carry):
                gb = g * LANES
                av = idxa_v[pl.ds(gb, LANES)]
                rv = idxr_v[pl.ds(gb, LANES)]
                nv = idxn_v[pl.ds(gb, LANES)]
                ta = plsc.load_gather(target_v, [av])
                tr = plsc.load_gather(target_v, [rv])
                tn = plsc.load_gather(target_v, [nv])
                fidx_v[pl.ds(gb, LANES)] = av * C + tr
                fidx_v[pl.ds(K + gb, LANES)] = rv * C + ta
                fidx_v[pl.ds(2 * K + gb, LANES)] = av * C + tn
                fidx_v[pl.ds(3 * K + gb, LANES)] = nv * C + ta
                return carry

            lax.fori_loop(0, GROUPS, group_a, 0)

            cpc = pltpu.async_copy(conf_flat.at[fidx_v], cval_v, sem)
            for cp in cps:
                cp.wait()
            cpc.wait()

            def group_b(g, acc_in):
                gb = g * LANES
                jv = gb + iota
                av = idxa_v[pl.ds(gb, LANES)]
                rv = idxr_v[pl.ds(gb, LANES)]
                nv = idxn_v[pl.ds(gb, LANES)]
                la = plsc.load_gather(lse_v, [av])
                lr = plsc.load_gather(lse_v, [rv])
                ln = plsc.load_gather(lse_v, [nv])
                c1 = cval_v[pl.ds(gb, LANES)]
                c2 = cval_v[pl.ds(K + gb, LANES)]
                c3 = cval_v[pl.ds(2 * K + gb, LANES)]
                c4 = cval_v[pl.ds(3 * K + gb, LANES)]
                w_rel = jnp.exp(jnp.exp(c1 - la) + jnp.exp(c2 - lr))
                w_neg = jnp.exp(jnp.exp(c3 - la) + jnp.exp(c4 - ln))

                def dbody(dd, c):
                    dap, dar, dan, dv = c
                    for _ in range(4):
                        cv = jnp.bitwise_and(dv + iota, D - 1)
                        ea = plsc.load_gather(ra_v, [jv, cv])
                        ep = plsc.load_gather(rp_v, [jv, cv])
                        er = plsc.load_gather(rr_v, [jv, cv])
                        en = plsc.load_gather(rn_v, [jv, cv])
                        s1 = ea - ep
                        s2 = ea - er
                        s3 = ea - en
                        dap = dap + s1 * s1
                        dar = dar + s2 * s2
                        dan = dan + s3 * s3
                        dv = dv + 1
                    return (dap, dar, dan, dv)

                dap, dar, dan, _ = lax.fori_loop(
                    0, D // 4, dbody,
                    (zf, zf, zf, jnp.zeros((LANES,), jnp.int32)))
                loss = (jnp.maximum(dap - dar + w_rel * MARGIN1, 0.0)
                        + jnp.maximum(dar - dan + w_neg * MARGIN2, 0.0))
                return acc_in + loss

            return lax.fori_loop(0, GROUPS, group_b, acc)

        acc = lax.fori_loop(0, NCHUNK, chunk_body, zf)
        accv[...] = acc
        pltpu.sync_copy(accv, out.at[wid])

    return sc_main


def kernel(embeddings, confidence, target, triplets):
    B, D = embeddings.shape
    C = confidence.shape[1]
    T = triplets.shape[0]

    lse = _make_lse(B, C, 512)(confidence)

    a_idx = triplets[:, 0]
    p_idx = triplets[:, 1]
    r_idx = triplets[:, 2]
    n_idx = triplets[:, 3]
    conf_flat = confidence.reshape(-1)

    partials = _make_sc_main(B, C, D, T, 256)(
        embeddings, a_idx, p_idx, r_idx, n_idx, target, lse, conf_flat)
    mean = jnp.sum(partials) / jnp.float32(T)
    return (mean, jnp.asarray(T, dtype=jnp.int32))


# speedup vs baseline: 11.9514x; 1.1249x over previous
import functools

import jax
import jax.numpy as jnp
from jax import lax
from jax.experimental import pallas as pl
from jax.experimental.pallas import tpu as pltpu
from jax.experimental.pallas import tpu_sc as plsc

MARGIN1 = 0.4
MARGIN2 = 0.4

NC = 2
NS = 16
LANES = 16
NW = NC * NS


def _make_lse(B, C, RB):
    def body(c_ref, o_ref):
        x = c_ref[...]
        m = jnp.max(x, axis=1)
        s = jnp.sum(jnp.exp(x - m[:, None]), axis=1)
        o_ref[...] = m + jnp.log(s)

    return pl.pallas_call(
        body,
        grid=(B // RB,),
        in_specs=[pl.BlockSpec((RB, C), lambda i: (i, 0))],
        out_specs=pl.BlockSpec((RB,), lambda i: (i,)),
        out_shape=jax.ShapeDtypeStruct((B,), jnp.float32),
    )


def _make_sc_main(B, C, D, T, K):
    TW = T // NW
    NCHUNK = TW // K
    GROUPS = K // LANES

    mesh = plsc.VectorSubcoreMesh(core_axis_name="c", subcore_axis_name="s")

    buf_t = [
        pltpu.VMEM((K,), jnp.int32),
        pltpu.VMEM((K,), jnp.int32),
        pltpu.VMEM((K,), jnp.int32),
        pltpu.VMEM((K,), jnp.int32),
        pltpu.VMEM((K, D), jnp.float32),
        pltpu.VMEM((K, D), jnp.float32),
        pltpu.VMEM((K, D), jnp.float32),
        pltpu.VMEM((K, D), jnp.float32),
        pltpu.VMEM((4 * K,), jnp.int32),
        pltpu.VMEM((4 * K,), jnp.float32),
    ]

    @functools.partial(
        pl.kernel,
        mesh=mesh,
        compiler_params=pltpu.CompilerParams(
            needs_layout_passes=False, use_tc_tiling_on_sc=False),
        out_type=jax.ShapeDtypeStruct((NW, LANES), jnp.float32),
        scratch_types=[
            pltpu.VMEM((B,), jnp.int32),
            pltpu.VMEM((B,), jnp.float32),
            *buf_t,
            *buf_t,
            pltpu.VMEM((LANES,), jnp.float32),
            pltpu.SemaphoreType.DMA,
            pltpu.SemaphoreType.DMA,
            pltpu.SemaphoreType.DMA,
        ],
    )
    def sc_main(emb, aidx, pidx, ridx, nidx, tgt, lse, conf_flat, out,
                target_v, lse_v, *rest):
        bufs = (rest[0:10], rest[10:20])
        accv = rest[20]
        sem_i = rest[21]
        sem_d = (rest[22], rest[23])

        wid = lax.axis_index("s") * NC + lax.axis_index("c")
        base_t = wid * TW

        pltpu.sync_copy(tgt, target_v)
        pltpu.sync_copy(lse, lse_v)

        zf = jnp.zeros((LANES,), jnp.float32)
        iota = lax.iota(jnp.int32, LANES)
        cols = (aidx, pidx, ridx, nidx)

        def fire_idx(ch, p):
            tb = base_t + ch * K
            for q in range(4):
                pltpu.async_copy(cols[q].at[pl.ds(tb, K)], bufs[p][q], sem_i)

        def drain_idx(p):
            for q in range(4):
                pltpu.make_async_copy(
                    cols[q].at[pl.ds(0, K)], bufs[p][q], sem_i).wait()

        def phase_a(p):
            idxa_v, _, idxr_v, idxn_v = bufs[p][0:4]
            fidx_v = bufs[p][8]

            def ga(g, carry):
                gb = g * LANES
                av = idxa_v[pl.ds(gb, LANES)]
                rv = idxr_v[pl.ds(gb, LANES)]
                nv = idxn_v[pl.ds(gb, LANES)]
                ta = plsc.load_gather(target_v, [av])
                tr = plsc.load_gather(target_v, [rv])
                tn = plsc.load_gather(target_v, [nv])
                fidx_v[pl.ds(gb, LANES)] = av * C + tr
                fidx_v[pl.ds(K + gb, LANES)] = rv * C + ta
                fidx_v[pl.ds(2 * K + gb, LANES)] = av * C + tn
                fidx_v[pl.ds(3 * K + gb, LANES)] = nv * C + ta
                return carry

            lax.fori_loop(0, GROUPS, ga, 0)

        def fire_rows(p):
            for q in range(4):
                pltpu.async_copy(emb.at[bufs[p][q]], bufs[p][4 + q], sem_d[p])
            pltpu.async_copy(conf_flat.at[bufs[p][8]], bufs[p][9], sem_d[p])

        def drain_rows(p):
            for q in range(4):
                pltpu.make_async_copy(
                    emb.at[bufs[p][q]], bufs[p][4 + q], sem_d[p]).wait()
            pltpu.make_async_copy(
                conf_flat.at[bufs[p][8]], bufs[p][9], sem_d[p]).wait()

        def phase_b(p, acc):
            idxa_v, _, idxr_v, idxn_v = bufs[p][0:4]
            ra_v, rp_v, rr_v, rn_v = bufs[p][4:8]
            cval_v = bufs[p][9]

            def gb_fn(g, acc_in):
                gb = g * LANES
                jv = gb + iota
                av = idxa_v[pl.ds(gb, LANES)]
                rv = idxr_v[pl.ds(gb, LANES)]
                nv = idxn_v[pl.ds(gb, LANES)]
                la = plsc.load_gather(lse_v, [av])
                lr = plsc.load_gather(lse_v, [rv])
                ln = plsc.load_gather(lse_v, [nv])
                c1 = cval_v[pl.ds(gb, LANES)]
                c2 = cval_v[pl.ds(K + gb, LANES)]
                c3 = cval_v[pl.ds(2 * K + gb, LANES)]
                c4 = cval_v[pl.ds(3 * K + gb, LANES)]
                w_rel = jnp.exp(jnp.exp(c1 - la) + jnp.exp(c2 - lr))
                w_neg = jnp.exp(jnp.exp(c3 - la) + jnp.exp(c4 - ln))

                def dbody(dd, c):
                    dap, dar, dan, dv = c
                    for _ in range(8):
                        cv = jnp.bitwise_and(dv + iota, D - 1)
                        ea = plsc.load_gather(ra_v, [jv, cv])
                        ep = plsc.load_gather(rp_v, [jv, cv])
                        er = plsc.load_gather(rr_v, [jv, cv])
                        en = plsc.load_gather(rn_v, [jv, cv])
                        s1 = ea - ep
                        s2 = ea - er
                        s3 = ea - en
                        dap = dap + s1 * s1
                        dar = dar + s2 * s2
                        dan = dan + s3 * s3
                        dv = dv + 1
                    return (dap, dar, dan, dv)

                dap, dar, dan, _ = lax.fori_loop(
                    0, D // 8, dbody,
                    (zf, zf, zf, jnp.zeros((LANES,), jnp.int32)))
                loss = (jnp.maximum(dap - dar + w_rel * MARGIN1, 0.0)
                        + jnp.maximum(dar - dan + w_neg * MARGIN2, 0.0))
                return acc_in + loss

            return lax.fori_loop(0, GROUPS, gb_fn, acc)

        fire_idx(0, 0)
        drain_idx(0)
        phase_a(0)
        fire_rows(0)

        def pair_body(gp, acc):
            e = 2 * gp
            o = e + 1
            e2 = jnp.minimum(e + 2, NCHUNK - 2)

            fire_idx(o, 1)
            drain_idx(1)
            phase_a(1)
            fire_rows(1)

            drain_rows(0)
            acc = phase_b(0, acc)

            fire_idx(e2, 0)
            drain_idx(0)
            phase_a(0)
            fire_rows(0)

            drain_rows(1)
            acc = phase_b(1, acc)
            return acc

        acc = lax.fori_loop(0, NCHUNK // 2, pair_body, zf)
        drain_rows(0)

        accv[...] = acc
        pltpu.sync_copy(accv, out.at[wid])

    return sc_main


def kernel(embeddings, confidence, target, triplets):
    B, D = embeddings.shape
    C = confidence.shape[1]
    T = triplets.shape[0]

    lse = _make_lse(B, C, 512)(confidence)

    a_idx = triplets[:, 0]
    p_idx = triplets[:, 1]
    r_idx = triplets[:, 2]
    n_idx = triplets[:, 3]
    conf_flat = confidence.reshape(-1)

    partials = _make_sc_main(B, C, D, T, 128)(
        embeddings, a_idx, p_idx, r_idx, n_idx, target, lse, conf_flat)
    mean = jnp.sum(partials) / jnp.float32(T)
    return (mean, jnp.asarray(T, dtype=jnp.int32))


# speedup vs baseline: 15.6275x; 1.3076x over previous
import functools

import jax
import jax.numpy as jnp
from jax import lax
from jax.experimental import pallas as pl
from jax.experimental.pallas import tpu as pltpu
from jax.experimental.pallas import tpu_sc as plsc

MARGIN1 = 0.4
MARGIN2 = 0.4

NC = 2
NS = 16
LANES = 16
NW = NC * NS


def _make_lse(B, C, RB):
    CP = 1024

    def body(c_ref, o_ref, flat_ref):
        x = c_ref[...]
        m = jnp.max(x, axis=1)
        s = jnp.sum(jnp.exp(x - m[:, None]), axis=1)
        o_ref[...] = m + jnp.log(s)
        xp = jnp.concatenate(
            [x, jnp.zeros((RB, CP - C), jnp.float32)], axis=1)
        flat_ref[...] = xp.reshape(RB * (CP // 128), 128)

    return pl.pallas_call(
        body,
        grid=(B // RB,),
        in_specs=[pl.BlockSpec((RB, C), lambda i: (i, 0))],
        out_specs=[pl.BlockSpec((RB,), lambda i: (i,)),
                   pl.BlockSpec((RB * (CP // 128), 128), lambda i: (i, 0))],
        out_shape=[jax.ShapeDtypeStruct((B,), jnp.float32),
                   jax.ShapeDtypeStruct((B * (CP // 128), 128), jnp.float32)],
    )


def _make_sc_main(B, CP, D, T, K):
    TW = T // NW
    NCHUNK = TW // K
    GROUPS = K // LANES

    mesh = plsc.VectorSubcoreMesh(core_axis_name="c", subcore_axis_name="s")

    buf_t = [
        pltpu.VMEM((K,), jnp.int32),
        pltpu.VMEM((K,), jnp.int32),
        pltpu.VMEM((K,), jnp.int32),
        pltpu.VMEM((K,), jnp.int32),
        pltpu.VMEM((K, D), jnp.float32),
        pltpu.VMEM((K, D), jnp.float32),
        pltpu.VMEM((K, D), jnp.float32),
        pltpu.VMEM((K, D), jnp.float32),
        pltpu.VMEM((4 * K,), jnp.int32),
        pltpu.VMEM((4 * K,), jnp.float32),
    ]

    @functools.partial(
        pl.kernel,
        mesh=mesh,
        compiler_params=pltpu.CompilerParams(
            needs_layout_passes=False, use_tc_tiling_on_sc=False),
        out_type=jax.ShapeDtypeStruct((NW, LANES), jnp.float32),
        scratch_types=[
            pltpu.VMEM((B,), jnp.int32),
            pltpu.VMEM((B,), jnp.float32),
            *buf_t,
            *buf_t,
            pltpu.VMEM((LANES,), jnp.float32),
            pltpu.SemaphoreType.DMA,
            pltpu.SemaphoreType.DMA,
            pltpu.SemaphoreType.DMA,
        ],
    )
    def sc_main(emb, aidx, pidx, ridx, nidx, tgt, lse, conf_flat, out,
                target_v, lse_v, *rest):
        bufs = (rest[0:10], rest[10:20])
        accv = rest[20]
        sem_i = rest[21]
        sem_d = (rest[22], rest[23])

        wid = lax.axis_index("s") * NC + lax.axis_index("c")
        base_t = wid * TW

        pltpu.sync_copy(tgt, target_v)
        pltpu.sync_copy(lse, lse_v)

        zf = jnp.zeros((LANES,), jnp.float32)
        iota = lax.iota(jnp.int32, LANES)
        cols = (aidx, pidx, ridx, nidx)

        def fire_idx(ch, p):
            tb = base_t + ch * K
            for q in range(4):
                pltpu.async_copy(cols[q].at[pl.ds(tb, K)], bufs[p][q], sem_i)

        def drain_idx(p):
            for q in range(4):
                pltpu.make_async_copy(
                    cols[q].at[pl.ds(0, K)], bufs[p][q], sem_i).wait()

        def phase_a(p):
            idxa_v, _, idxr_v, idxn_v = bufs[p][0:4]
            fidx_v = bufs[p][8]

            def ga(g, carry):
                gb = g * LANES
                av = idxa_v[pl.ds(gb, LANES)]
                rv = idxr_v[pl.ds(gb, LANES)]
                nv = idxn_v[pl.ds(gb, LANES)]
                ta = plsc.load_gather(target_v, [av])
                tr = plsc.load_gather(target_v, [rv])
                tn = plsc.load_gather(target_v, [nv])
                fidx_v[pl.ds(gb, LANES)] = av * CP + tr
                fidx_v[pl.ds(K + gb, LANES)] = rv * CP + ta
                fidx_v[pl.ds(2 * K + gb, LANES)] = av * CP + tn
                fidx_v[pl.ds(3 * K + gb, LANES)] = nv * CP + ta
                return carry

            lax.fori_loop(0, GROUPS, ga, 0)

        def fire_rows(p):
            for q in range(4):
                pltpu.async_copy(emb.at[bufs[p][q]], bufs[p][4 + q], sem_d[p])
            pltpu.async_copy(conf_flat.at[bufs[p][8]], bufs[p][9], sem_d[p])

        def drain_rows(p):
            for q in range(4):
                pltpu.make_async_copy(
                    emb.at[bufs[p][q]], bufs[p][4 + q], sem_d[p]).wait()
            pltpu.make_async_copy(
                conf_flat.at[bufs[p][8]], bufs[p][9], sem_d[p]).wait()

        def phase_b(p, acc):
            idxa_v, _, idxr_v, idxn_v = bufs[p][0:4]
            ra_v, rp_v, rr_v, rn_v = bufs[p][4:8]
            cval_v = bufs[p][9]

            def gb_fn(g, acc_in):
                gb = g * LANES
                jv = gb + iota
                av = idxa_v[pl.ds(gb, LANES)]
                rv = idxr_v[pl.ds(gb, LANES)]
                nv = idxn_v[pl.ds(gb, LANES)]
                la = plsc.load_gather(lse_v, [av])
                lr = plsc.load_gather(lse_v, [rv])
                ln = plsc.load_gather(lse_v, [nv])
                c1 = cval_v[pl.ds(gb, LANES)]
                c2 = cval_v[pl.ds(K + gb, LANES)]
                c3 = cval_v[pl.ds(2 * K + gb, LANES)]
                c4 = cval_v[pl.ds(3 * K + gb, LANES)]
                w_rel = jnp.exp(jnp.exp(c1 - la) + jnp.exp(c2 - lr))
                w_neg = jnp.exp(jnp.exp(c3 - la) + jnp.exp(c4 - ln))

                def dbody(dd, c):
                    dap, dar, dan, dv = c
                    for _ in range(8):
                        cv = jnp.bitwise_and(dv + iota, D - 1)
                        ea = plsc.load_gather(ra_v, [jv, cv])
                        ep = plsc.load_gather(rp_v, [jv, cv])
                        er = plsc.load_gather(rr_v, [jv, cv])
                        en = plsc.load_gather(rn_v, [jv, cv])
                        s1 = ea - ep
                        s2 = ea - er
                        s3 = ea - en
                        dap = dap + s1 * s1
                        dar = dar + s2 * s2
                        dan = dan + s3 * s3
                        dv = dv + 1
                    return (dap, dar, dan, dv)

                dap, dar, dan, _ = lax.fori_loop(
                    0, D // 8, dbody,
                    (zf, zf, zf, jnp.zeros((LANES,), jnp.int32)))
                loss = (jnp.maximum(dap - dar + w_rel * MARGIN1, 0.0)
                        + jnp.maximum(dar - dan + w_neg * MARGIN2, 0.0))
                return acc_in + loss

            return lax.fori_loop(0, GROUPS, gb_fn, acc)

        fire_idx(0, 0)
        drain_idx(0)
        phase_a(0)
        fire_rows(0)

        def pair_body(gp, acc):
            e = 2 * gp
            o = e + 1
            e2 = jnp.minimum(e + 2, NCHUNK - 2)

            fire_idx(o, 1)
            drain_idx(1)
            phase_a(1)
            fire_rows(1)

            drain_rows(0)
            acc = phase_b(0, acc)

            fire_idx(e2, 0)
            drain_idx(0)
            phase_a(0)
            fire_rows(0)

            drain_rows(1)
            acc = phase_b(1, acc)
            return acc

        acc = lax.fori_loop(0, NCHUNK // 2, pair_body, zf)
        drain_rows(0)

        accv[...] = acc
        pltpu.sync_copy(accv, out.at[wid])

    return sc_main


def kernel(embeddings, confidence, target, triplets):
    B, D = embeddings.shape
    C = confidence.shape[1]
    T = triplets.shape[0]

    lse, conf_pad = _make_lse(B, C, 512)(confidence)
    conf_flat = conf_pad.reshape(-1)

    a_idx = triplets[:, 0]
    p_idx = triplets[:, 1]
    r_idx = triplets[:, 2]
    n_idx = triplets[:, 3]

    partials = _make_sc_main(B, 1024, D, T, 128)(
        embeddings, a_idx, p_idx, r_idx, n_idx, target, lse, conf_flat)
    mean = jnp.sum(partials) / jnp.float32(T)
    return (mean, jnp.asarray(T, dtype=jnp.int32))
